# baseline (speedup 1.0000x reference)
import functools

import jax
import jax.numpy as jnp
from jax import lax
from jax.experimental import pallas as pl
from jax.experimental.pallas import tpu as pltpu
from jax.experimental.pallas import tpu_sc as plsc

_B = 16384
_D = 768
_NP = 512
_NQ = 64


def _table_body(c0_ref, c1_ref, c2_ref, c3_ref, c4_ref,
                out_ref, t2_ref, m_ref, mask_ref):
    q = pl.program_id(0)

    @pl.when(q == 0)
    def _prep():
        t1 = jnp.dot(c0_ref[...], c1_ref[...],
                     preferred_element_type=jnp.float32)
        t1 = (t1.reshape(8, 2, 8, 4, 64).transpose(0, 2, 1, 3, 4)
                .reshape(512, 64))
        t2 = jnp.dot(t1, c2_ref[...],
                     preferred_element_type=jnp.float32)
        t2 = (t2.reshape(64, 8, 8, 6, 64).transpose(0, 2, 1, 3, 4)
                .reshape(_NP, 3072))
        t2_ref[...] = t2.astype(jnp.bfloat16)

        m = jnp.dot(c3_ref[...], c4_ref[...],
                    preferred_element_type=jnp.float32)
        m = (m.reshape(8, 64, 4, 8, 4).transpose(0, 3, 1, 2, 4)
              .reshape(_NQ, _NQ, 16))
        m_ref[...] = m.astype(jnp.bfloat16)

        mask_ref[...] = jnp.zeros((1024, 256), jnp.bfloat16)
        for d in range(16):
            mask_ref[d * 64:(d + 1) * 64, d * 16:(d + 1) * 16] = jnp.ones(
                (64, 16), jnp.bfloat16)

    mq = m_ref[q]
    rhs = jnp.tile(mq, (16, 16)) * mask_ref[...]
    lhs = t2_ref[...]
    for g in range(3):
        out_ref[:, g * 256:(g + 1) * 256] = jnp.dot(
            lhs[:, g * 1024:(g + 1) * 1024], rhs,
            preferred_element_type=jnp.float32)


_SC_CHUNK = 64
_N_WORKERS = 32
_PER_W = _B // _N_WORKERS
_N_CHUNKS = _PER_W // _SC_CHUNK


def _make_gather():
    mesh = plsc.VectorSubcoreMesh(core_axis_name="c", subcore_axis_name="s")

    @functools.partial(
        pl.kernel, mesh=mesh,
        out_type=jax.ShapeDtypeStruct((_B, _D), jnp.float32),
        scratch_types=[
            pltpu.VMEM((_SC_CHUNK,), jnp.int32),
            pltpu.VMEM((_SC_CHUNK,), jnp.int32),
            pltpu.VMEM((_SC_CHUNK, _D), jnp.float32),
            pltpu.VMEM((_SC_CHUNK, _D), jnp.float32),
            pltpu.SemaphoreType.DMA,
            pltpu.SemaphoreType.DMA,
        ],
    )
    def gather_kernel(tab_hbm, idx_hbm, out_hbm,
                      idx0, idx1, rows0, rows1, sem0, sem1):
        wid = lax.axis_index("s") * 2 + lax.axis_index("c")
        base = wid * _PER_W
        bufs = ((idx0, rows0, sem0), (idx1, rows1, sem1))

        def fetch(idx_v, rows_v, sem, off):
            pltpu.sync_copy(idx_hbm.at[pl.ds(off, _SC_CHUNK)], idx_v)
            for k in range(_SC_CHUNK // 16):
                v = idx_v[pl.ds(k * 16, 16)]
                idx_v[pl.ds(k * 16, 16)] = (v & 63) * _NP + (v >> 6)
            pltpu.async_copy(tab_hbm.at[idx_v], rows_v, sem)

        fetch(idx0, rows0, sem0, base)
        for c in range(_N_CHUNKS):
            idx_c, rows_c, sem_c = bufs[c % 2]
            if c + 1 < _N_CHUNKS:
                idx_n, rows_n, sem_n = bufs[(c + 1) % 2]
                fetch(idx_n, rows_n, sem_n, base + (c + 1) * _SC_CHUNK)
            pltpu.make_async_copy(tab_hbm.at[idx_c], rows_c, sem_c).wait()
            pltpu.sync_copy(rows_c, out_hbm.at[pl.ds(base + c * _SC_CHUNK,
                                                     _SC_CHUNK)])

    return gather_kernel


def kernel(core0, core1, core2, core3, core4, input):
    c0f = core0.reshape(16, 16)
    c1f = core1.reshape(16, 2048)
    c2f = core2.reshape(64, 3072)
    c3f = core3.transpose(1, 0, 2, 3).reshape(2048, 32)
    c4f = core4.reshape(32, 32)
    idxf = input.astype(jnp.int32).reshape(_B)

    table = pl.pallas_call(
        _table_body,
        grid=(_NQ,),
        in_specs=[
            pl.BlockSpec((16, 16), lambda q: (0, 0)),
            pl.BlockSpec((16, 2048), lambda q: (0, 0)),
            pl.BlockSpec((64, 3072), lambda q: (0, 0)),
            pl.BlockSpec((2048, 32), lambda q: (0, 0)),
            pl.BlockSpec((32, 32), lambda q: (0, 0)),
        ],
        out_specs=pl.BlockSpec((_NP, _D), lambda q: (q, 0)),
        out_shape=jax.ShapeDtypeStruct((_NQ * _NP, _D), jnp.float32),
        scratch_shapes=[
            pltpu.VMEM((_NP, 3072), jnp.bfloat16),
            pltpu.VMEM((_NQ, _NQ, 16), jnp.bfloat16),
            pltpu.VMEM((1024, 256), jnp.bfloat16),
        ],
    )(c0f, c1f, c2f, c3f, c4f)

    out = _make_gather()(table, idxf)
    return out.reshape(4, 4096, _D)


# speedup vs baseline: 1.0699x; 1.0699x over previous
import functools

import jax
import jax.numpy as jnp
from jax import lax
from jax.experimental import pallas as pl
from jax.experimental.pallas import tpu as pltpu
from jax.experimental.pallas import tpu_sc as plsc

_B = 16384
_D = 768
_NP = 512
_NQ = 64
_QB = 2


def _table_body(c0_ref, c1_ref, c2_ref, c3_ref, c4_ref,
                out_ref, t2_ref, m_ref, mask_ref):
    q = pl.program_id(0)

    @pl.when(q == 0)
    def _prep():
        t1 = jnp.dot(c0_ref[...], c1_ref[...],
                     preferred_element_type=jnp.float32)
        t1 = (t1.reshape(8, 2, 8, 4, 64).transpose(0, 2, 1, 3, 4)
                .reshape(512, 64))
        t2 = jnp.dot(t1, c2_ref[...],
                     preferred_element_type=jnp.float32)
        t2 = (t2.reshape(64, 8, 8, 6, 64).transpose(0, 2, 1, 3, 4)
                .reshape(_NP, 3072))
        t2_ref[...] = t2.astype(jnp.bfloat16)

        m = jnp.dot(c3_ref[...], c4_ref[...],
                    preferred_element_type=jnp.float32)
        m = (m.reshape(8, 64, 4, 8, 4).transpose(0, 3, 1, 2, 4)
              .reshape(_NQ, _NQ, 16))
        m_ref[...] = m.astype(jnp.bfloat16)

        mask_ref[...] = jnp.zeros((1024, 256), jnp.bfloat16)
        for d in range(16):
            mask_ref[d * 64:(d + 1) * 64, d * 16:(d + 1) * 16] = jnp.ones(
                (64, 16), jnp.bfloat16)

    lhs = t2_ref[...]
    for h in range(_QB):
        mq = m_ref[q * _QB + h]
        rhs = jnp.tile(mq, (16, 16)) * mask_ref[...]
        for g in range(3):
            out_ref[h * _NP:(h + 1) * _NP, g * 256:(g + 1) * 256] = jnp.dot(
                lhs[:, g * 1024:(g + 1) * 1024], rhs,
                preferred_element_type=jnp.float32)


_SC_CHUNK = 64
_N_WORKERS = 32
_PER_W = _B // _N_WORKERS
_N_CHUNKS = _PER_W // _SC_CHUNK


def _make_gather():
    mesh = plsc.VectorSubcoreMesh(core_axis_name="c", subcore_axis_name="s")

    @functools.partial(
        pl.kernel, mesh=mesh,
        out_type=jax.ShapeDtypeStruct((_B, _D), jnp.float32),
        scratch_types=[
            pltpu.VMEM((_SC_CHUNK,), jnp.int32),
            pltpu.VMEM((_SC_CHUNK,), jnp.int32),
            pltpu.VMEM((_SC_CHUNK, _D), jnp.float32),
            pltpu.VMEM((_SC_CHUNK, _D), jnp.float32),
            pltpu.SemaphoreType.DMA,
            pltpu.SemaphoreType.DMA,
        ],
    )
    def gather_kernel(tab_hbm, idx_hbm, out_hbm,
                      idx0, idx1, rows0, rows1, sem0, sem1):
        wid = lax.axis_index("s") * 2 + lax.axis_index("c")
        base = wid * _PER_W
        bufs = ((idx0, rows0, sem0), (idx1, rows1, sem1))

        def fetch(idx_v, rows_v, sem, off):
            pltpu.sync_copy(idx_hbm.at[pl.ds(off, _SC_CHUNK)], idx_v)
            for k in range(_SC_CHUNK // 16):
                v = idx_v[pl.ds(k * 16, 16)]
                idx_v[pl.ds(k * 16, 16)] = (v & 63) * _NP + (v >> 6)
            pltpu.async_copy(tab_hbm.at[idx_v], rows_v, sem)

        fetch(idx0, rows0, sem0, base)
        for c in range(_N_CHUNKS):
            idx_c, rows_c, sem_c = bufs[c % 2]
            if c + 1 < _N_CHUNKS:
                idx_n, rows_n, sem_n = bufs[(c + 1) % 2]
                fetch(idx_n, rows_n, sem_n, base + (c + 1) * _SC_CHUNK)
            pltpu.make_async_copy(tab_hbm.at[idx_c], rows_c, sem_c).wait()
            pltpu.sync_copy(rows_c, out_hbm.at[pl.ds(base + c * _SC_CHUNK,
                                                     _SC_CHUNK)])

    return gather_kernel


def kernel(core0, core1, core2, core3, core4, input):
    c0f = core0.reshape(16, 16)
    c1f = core1.reshape(16, 2048)
    c2f = core2.reshape(64, 3072)
    c3f = core3.transpose(1, 0, 2, 3).reshape(2048, 32)
    c4f = core4.reshape(32, 32)
    idxf = input.astype(jnp.int32).reshape(_B)

    table = pl.pallas_call(
        _table_body,
        grid=(_NQ // _QB,),
        in_specs=[
            pl.BlockSpec((16, 16), lambda q: (0, 0)),
            pl.BlockSpec((16, 2048), lambda q: (0, 0)),
            pl.BlockSpec((64, 3072), lambda q: (0, 0)),
            pl.BlockSpec((2048, 32), lambda q: (0, 0)),
            pl.BlockSpec((32, 32), lambda q: (0, 0)),
        ],
        out_specs=pl.BlockSpec((_QB * _NP, _D), lambda q: (q, 0)),
        out_shape=jax.ShapeDtypeStruct((_NQ * _NP, _D), jnp.float32),
        scratch_shapes=[
            pltpu.VMEM((_NP, 3072), jnp.bfloat16),
            pltpu.VMEM((_NQ, _NQ, 16), jnp.bfloat16),
            pltpu.VMEM((1024, 256), jnp.bfloat16),
        ],
    )(c0f, c1f, c2f, c3f, c4f)

    out = _make_gather()(table, idxf)
    return out.reshape(4, 4096, _D)


# speedup vs baseline: 1.1059x; 1.0336x over previous
import functools

import jax
import jax.numpy as jnp
from jax import lax
from jax.experimental import pallas as pl
from jax.experimental.pallas import tpu as pltpu
from jax.experimental.pallas import tpu_sc as plsc

_B = 16384
_D = 768
_NP = 512
_NQ = 64
_QB = 4


def _table_body(c0_ref, c1_ref, c2_ref, c3_ref, c4_ref,
                out_ref, t2_ref, m_ref, mask_ref):
    q = pl.program_id(0)

    @pl.when(q == 0)
    def _prep():
        t1 = jnp.dot(c0_ref[...], c1_ref[...],
                     preferred_element_type=jnp.float32)
        t1 = (t1.reshape(8, 2, 8, 4, 64).transpose(0, 2, 1, 3, 4)
                .reshape(512, 64))
        t2 = jnp.dot(t1, c2_ref[...],
                     preferred_element_type=jnp.float32)
        t2 = (t2.reshape(64, 8, 8, 6, 64).transpose(0, 2, 1, 3, 4)
                .reshape(_NP, 3072))
        t2_ref[...] = t2.astype(jnp.bfloat16)

        m = jnp.dot(c3_ref[...], c4_ref[...],
                    preferred_element_type=jnp.float32)
        m = (m.reshape(8, 64, 4, 8, 4).transpose(0, 3, 1, 2, 4)
              .reshape(_NQ, _NQ, 16))
        m_ref[...] = m.astype(jnp.bfloat16)

        mask_ref[...] = jnp.zeros((1024, 256), jnp.bfloat16)
        for d in range(16):
            mask_ref[d * 64:(d + 1) * 64, d * 16:(d + 1) * 16] = jnp.ones(
                (64, 16), jnp.bfloat16)

    lhs = t2_ref[...]
    for h in range(_QB):
        mq = m_ref[q * _QB + h]
        rhs = jnp.tile(mq, (16, 16)) * mask_ref[...]
        for g in range(3):
            out_ref[h * _NP:(h + 1) * _NP, g * 256:(g + 1) * 256] = jnp.dot(
                lhs[:, g * 1024:(g + 1) * 1024], rhs,
                preferred_element_type=jnp.float32)


_SC_CHUNK = 64
_N_WORKERS = 32
_PER_W = _B // _N_WORKERS
_N_CHUNKS = _PER_W // _SC_CHUNK


def _make_gather():
    mesh = plsc.VectorSubcoreMesh(core_axis_name="c", subcore_axis_name="s")

    @functools.partial(
        pl.kernel, mesh=mesh,
        out_type=jax.ShapeDtypeStruct((_B, _D), jnp.float32),
        scratch_types=[
            pltpu.VMEM((_SC_CHUNK,), jnp.int32),
            pltpu.VMEM((_SC_CHUNK,), jnp.int32),
            pltpu.VMEM((_SC_CHUNK, _D), jnp.float32),
            pltpu.VMEM((_SC_CHUNK, _D), jnp.float32),
            pltpu.SemaphoreType.DMA,
            pltpu.SemaphoreType.DMA,
        ],
    )
    def gather_kernel(tab_hbm, idx_hbm, out_hbm,
                      idx0, idx1, rows0, rows1, sem0, sem1):
        wid = lax.axis_index("s") * 2 + lax.axis_index("c")
        base = wid * _PER_W
        bufs = ((idx0, rows0, sem0), (idx1, rows1, sem1))

        def fetch(idx_v, rows_v, sem, off):
            pltpu.sync_copy(idx_hbm.at[pl.ds(off, _SC_CHUNK)], idx_v)
            for k in range(_SC_CHUNK // 16):
                v = idx_v[pl.ds(k * 16, 16)]
                idx_v[pl.ds(k * 16, 16)] = (v & 63) * _NP + (v >> 6)
            pltpu.async_copy(tab_hbm.at[idx_v], rows_v, sem)

        fetch(idx0, rows0, sem0, base)
        for c in range(_N_CHUNKS):
            idx_c, rows_c, sem_c = bufs[c % 2]
            if c + 1 < _N_CHUNKS:
                idx_n, rows_n, sem_n = bufs[(c + 1) % 2]
                fetch(idx_n, rows_n, sem_n, base + (c + 1) * _SC_CHUNK)
            pltpu.make_async_copy(tab_hbm.at[idx_c], rows_c, sem_c).wait()
            pltpu.sync_copy(rows_c, out_hbm.at[pl.ds(base + c * _SC_CHUNK,
                                                     _SC_CHUNK)])

    return gather_kernel


def kernel(core0, core1, core2, core3, core4, input):
    c0f = core0.reshape(16, 16)
    c1f = core1.reshape(16, 2048)
    c2f = core2.reshape(64, 3072)
    c3f = core3.transpose(1, 0, 2, 3).reshape(2048, 32)
    c4f = core4.reshape(32, 32)
    idxf = input.astype(jnp.int32).reshape(_B)

    table = pl.pallas_call(
        _table_body,
        grid=(_NQ // _QB,),
        in_specs=[
            pl.BlockSpec((16, 16), lambda q: (0, 0)),
            pl.BlockSpec((16, 2048), lambda q: (0, 0)),
            pl.BlockSpec((64, 3072), lambda q: (0, 0)),
            pl.BlockSpec((2048, 32), lambda q: (0, 0)),
            pl.BlockSpec((32, 32), lambda q: (0, 0)),
        ],
        out_specs=pl.BlockSpec((_QB * _NP, _D), lambda q: (q, 0)),
        out_shape=jax.ShapeDtypeStruct((_NQ * _NP, _D), jnp.float32),
        scratch_shapes=[
            pltpu.VMEM((_NP, 3072), jnp.bfloat16),
            pltpu.VMEM((_NQ, _NQ, 16), jnp.bfloat16),
            pltpu.VMEM((1024, 256), jnp.bfloat16),
        ],
    )(c0f, c1f, c2f, c3f, c4f)

    out = _make_gather()(table, idxf)
    return out.reshape(4, 4096, _D)


# speedup vs baseline: 1.1118x; 1.0054x over previous
import functools

import jax
import jax.numpy as jnp
from jax import lax
from jax.experimental import pallas as pl
from jax.experimental.pallas import tpu as pltpu
from jax.experimental.pallas import tpu_sc as plsc

_B = 16384
_D = 768
_NP = 512
_NQ = 64
_QB = 8


def _table_body(c0_ref, c1_ref, c2_ref, c3_ref, c4_ref,
                out_ref, t2_ref, m_ref, mask_ref):
    q = pl.program_id(0)

    @pl.when(q == 0)
    def _prep():
        t1 = jnp.dot(c0_ref[...], c1_ref[...],
                     preferred_element_type=jnp.float32)
        t1 = (t1.reshape(8, 2, 8, 4, 64).transpose(0, 2, 1, 3, 4)
                .reshape(512, 64))
        t2 = jnp.dot(t1, c2_ref[...],
                     preferred_element_type=jnp.float32)
        t2 = (t2.reshape(64, 8, 8, 6, 64).transpose(0, 2, 1, 3, 4)
                .reshape(_NP, 3072))
        t2_ref[...] = t2.astype(jnp.bfloat16)

        m = jnp.dot(c3_ref[...], c4_ref[...],
                    preferred_element_type=jnp.float32)
        m = (m.reshape(8, 64, 4, 8, 4).transpose(0, 3, 1, 2, 4)
              .reshape(_NQ, _NQ, 16))
        m_ref[...] = m.astype(jnp.bfloat16)

        mask_ref[...] = jnp.zeros((1024, 256), jnp.bfloat16)
        for d in range(16):
            mask_ref[d * 64:(d + 1) * 64, d * 16:(d + 1) * 16] = jnp.ones(
                (64, 16), jnp.bfloat16)

    lhs = t2_ref[...]
    for h in range(_QB):
        mq = m_ref[q * _QB + h]
        rhs = jnp.tile(mq, (16, 16)) * mask_ref[...]
        for g in range(3):
            out_ref[h * _NP:(h + 1) * _NP, g * 256:(g + 1) * 256] = jnp.dot(
                lhs[:, g * 1024:(g + 1) * 1024], rhs,
                preferred_element_type=jnp.float32)


_SC_CHUNK = 64
_N_WORKERS = 32
_PER_W = _B // _N_WORKERS
_N_CHUNKS = _PER_W // _SC_CHUNK


def _make_gather():
    mesh = plsc.VectorSubcoreMesh(core_axis_name="c", subcore_axis_name="s")

    @functools.partial(
        pl.kernel, mesh=mesh,
        out_type=jax.ShapeDtypeStruct((_B, _D), jnp.float32),
        scratch_types=[
            pltpu.VMEM((_SC_CHUNK,), jnp.int32),
            pltpu.VMEM((_SC_CHUNK,), jnp.int32),
            pltpu.VMEM((_SC_CHUNK, _D), jnp.float32),
            pltpu.VMEM((_SC_CHUNK, _D), jnp.float32),
            pltpu.SemaphoreType.DMA,
            pltpu.SemaphoreType.DMA,
        ],
    )
    def gather_kernel(tab_hbm, idx_hbm, out_hbm,
                      idx0, idx1, rows0, rows1, sem0, sem1):
        wid = lax.axis_index("s") * 2 + lax.axis_index("c")
        base = wid * _PER_W
        bufs = ((idx0, rows0, sem0), (idx1, rows1, sem1))

        def fetch(idx_v, rows_v, sem, off):
            pltpu.sync_copy(idx_hbm.at[pl.ds(off, _SC_CHUNK)], idx_v)
            for k in range(_SC_CHUNK // 16):
                v = idx_v[pl.ds(k * 16, 16)]
                idx_v[pl.ds(k * 16, 16)] = (v & 63) * _NP + (v >> 6)
            pltpu.async_copy(tab_hbm.at[idx_v], rows_v, sem)

        fetch(idx0, rows0, sem0, base)
        for c in range(_N_CHUNKS):
            idx_c, rows_c, sem_c = bufs[c % 2]
            if c + 1 < _N_CHUNKS:
                idx_n, rows_n, sem_n = bufs[(c + 1) % 2]
                fetch(idx_n, rows_n, sem_n, base + (c + 1) * _SC_CHUNK)
            pltpu.make_async_copy(tab_hbm.at[idx_c], rows_c, sem_c).wait()
            pltpu.sync_copy(rows_c, out_hbm.at[pl.ds(base + c * _SC_CHUNK,
                                                     _SC_CHUNK)])

    return gather_kernel


def kernel(core0, core1, core2, core3, core4, input):
    c0f = core0.reshape(16, 16)
    c1f = core1.reshape(16, 2048)
    c2f = core2.reshape(64, 3072)
    c3f = core3.transpose(1, 0, 2, 3).reshape(2048, 32)
    c4f = core4.reshape(32, 32)
    idxf = input.astype(jnp.int32).reshape(_B)

    table = pl.pallas_call(
        _table_body,
        grid=(_NQ // _QB,),
        in_specs=[
            pl.BlockSpec((16, 16), lambda q: (0, 0)),
            pl.BlockSpec((16, 2048), lambda q: (0, 0)),
            pl.BlockSpec((64, 3072), lambda q: (0, 0)),
            pl.BlockSpec((2048, 32), lambda q: (0, 0)),
            pl.BlockSpec((32, 32), lambda q: (0, 0)),
        ],
        out_specs=pl.BlockSpec((_QB * _NP, _D), lambda q: (q, 0)),
        out_shape=jax.ShapeDtypeStruct((_NQ * _NP, _D), jnp.float32),
        scratch_shapes=[
            pltpu.VMEM((_NP, 3072), jnp.bfloat16),
            pltpu.VMEM((_NQ, _NQ, 16), jnp.bfloat16),
            pltpu.VMEM((1024, 256), jnp.bfloat16),
        ],
    )(c0f, c1f, c2f, c3f, c4f)

    out = _make_gather()(table, idxf)
    return out.reshape(4, 4096, _D)
